# baseline (speedup 1.0000x reference)
import functools

import jax
import jax.numpy as jnp
from jax import lax
from jax.experimental import pallas as pl
from jax.experimental.pallas import tpu as pltpu
from jax.experimental.pallas import tpu_sc as plsc

N_NODES = 10000
F = 128
N_EDGES = 320000

NC = 2
NS = 16
NW = NC * NS
K = 128
CPW = 80
E_PAD = NW * CPW * K
PAD_DST_ROWS = 240

ACC_ROWS = 10240
ZPT = ACC_ROWS // NS
CNT_PAD = 10240
CPT = CNT_PAD // NS


def _agg_body(with_cnt, *refs):
    if with_cnt:
        (x_hbm, src_hbm, dst_hbm, out_hbm, cnt_hbm,
         acc_sh, cnt_sh, zflat, src1d, dstv0, dstv1, rows,
         onesv, gsem0, gsem1, dsem0, dsem1, ssem0, ssem1, csem0,
         csem1) = refs
    else:
        (x_hbm, src_hbm, dst_hbm, out_hbm,
         acc_sh, src1d, dstv0, dstv1, rows, gsem0, gsem1, dsem0,
         dsem1, ssem0, ssem1) = refs

    c = lax.axis_index("c")
    s = lax.axis_index("s")
    wid = c * NS + s
    ebase = wid * CPW * K

    pltpu.sync_copy(src_hbm.at[pl.ds(ebase, CPW * K)], src1d)

    zero16 = jnp.zeros((16,), jnp.float32)

    def _zrow(i, carry):
        for jj in range(8):
            rows[0, i, pl.ds(jj * 16, 16)] = zero16
        return carry

    lax.fori_loop(0, K, _zrow, 0)

    for b in range(ZPT // K):
        pltpu.sync_copy(rows.at[0],
                        acc_sh.at[pl.ds(s * ZPT + b * K, K)])

    if with_cnt:
        def _zflat(i, carry):
            zflat[pl.ds(i * 16, 16)] = zero16
            return carry

        lax.fori_loop(0, CPT // 16, _zflat, 0)
        pltpu.sync_copy(zflat, cnt_sh.at[pl.ds(s * CPT, CPT)])
        one16 = jnp.ones((16,), jnp.float32)
        for jj in range(K // 16):
            onesv[pl.ds(jj * 16, 16)] = one16

    def _start(j, b, dref, gsem, dsem):
        pltpu.async_copy(dst_hbm.at[pl.ds(ebase + j * K, K)], dref, dsem)
        pltpu.async_copy(x_hbm.at[src1d.at[pl.ds(j * K, K)]], rows.at[b],
                         gsem)

    _start(0, 0, dstv0, gsem0, dsem0)
    _start(1, 1, dstv1, gsem1, dsem1)

    plsc.subcore_barrier()

    def _fire(jd, b, dref, gsem, dsem, ssem, csem):
        pltpu.make_async_copy(dst_hbm.at[pl.ds(ebase + jd * K, K)], dref,
                              dsem).wait()
        pltpu.make_async_copy(x_hbm.at[src1d.at[pl.ds(jd * K, K)]],
                              rows.at[b], gsem).wait()
        pltpu.async_copy(rows.at[b], acc_sh.at[dref], ssem, add=True)
        if with_cnt:
            pltpu.async_copy(onesv, cnt_sh.at[dref], csem, add=True)

    def _drain(b, dref, ssem, csem):
        pltpu.make_async_copy(rows.at[b], acc_sh.at[dref], ssem).wait()
        if with_cnt:
            pltpu.make_async_copy(onesv, cnt_sh.at[dref], csem).wait()

    def _pair(i, carry):
        j0 = i * 2
        _fire(j0, 0, dstv0, gsem0, dsem0, ssem0, csem0 if with_cnt else None)
        _fire(j0 + 1, 1, dstv1, gsem1, dsem1, ssem1,
              csem1 if with_cnt else None)
        _drain(0, dstv0, ssem0, csem0 if with_cnt else None)

        @pl.when(j0 + 2 < CPW)
        def _n0():
            _start(j0 + 2, 0, dstv0, gsem0, dsem0)

        _drain(1, dstv1, ssem1, csem1 if with_cnt else None)

        @pl.when(j0 + 3 < CPW)
        def _n1():
            _start(j0 + 3, 1, dstv1, gsem1, dsem1)

        return carry

    lax.fori_loop(0, CPW // 2, _pair, 0)

    plsc.subcore_barrier()

    obase = c * N_NODES + s * ZPT

    @pl.when(s < NS - 1)
    def _copy_full():
        for b in range(ZPT // 128):
            pltpu.sync_copy(acc_sh.at[pl.ds(s * ZPT + b * 128, 128)],
                            out_hbm.at[pl.ds(obase + b * 128, 128)])

    @pl.when(s == NS - 1)
    def _copy_last():
        rem = N_NODES - (NS - 1) * ZPT
        for b in range(rem // 128):
            pltpu.sync_copy(acc_sh.at[pl.ds(s * ZPT + b * 128, 128)],
                            out_hbm.at[pl.ds(obase + b * 128, 128)])
        r = rem - (rem // 128) * 128
        if r:
            pltpu.sync_copy(
                acc_sh.at[pl.ds(s * ZPT + (rem // 128) * 128, r)],
                out_hbm.at[pl.ds(obase + (rem // 128) * 128, r)])
    if with_cnt:
        pltpu.sync_copy(cnt_sh.at[pl.ds(s * CPT, CPT)],
                        cnt_hbm.at[pl.ds(c * CNT_PAD + s * CPT, CPT)])


def _make_agg(with_cnt):
    mesh = plsc.VectorSubcoreMesh(core_axis_name="c", subcore_axis_name="s",
                                  num_cores=NC, num_subcores=NS)
    out_type = [jax.ShapeDtypeStruct((2 * N_NODES, F), jnp.float32)]
    scratch = [
        pltpu.VMEM_SHARED((ACC_ROWS, F), jnp.float32),
    ]
    if with_cnt:
        out_type.append(jax.ShapeDtypeStruct((2 * CNT_PAD,), jnp.float32))
        scratch.append(pltpu.VMEM_SHARED((CNT_PAD,), jnp.float32))
    if with_cnt:
        scratch.append(pltpu.VMEM((CPT,), jnp.float32))
    scratch += [
        pltpu.VMEM((CPW * K,), jnp.int32),
        pltpu.VMEM((K,), jnp.int32),
        pltpu.VMEM((K,), jnp.int32),
        pltpu.VMEM((2, K, F), jnp.float32),
    ]
    if with_cnt:
        scratch.append(pltpu.VMEM((K,), jnp.float32))
    scratch += [pltpu.SemaphoreType.DMA] * (8 if with_cnt else 6)
    return pl.kernel(functools.partial(_agg_body, with_cnt),
                     out_type=tuple(out_type) if with_cnt else out_type[0],
                     mesh=mesh, scratch_types=scratch)


B = 1000
GRID = N_NODES // B


def _combine_body(mode, parts_ref, cnt_ref, x_ref, wl_ref, bl_ref, wr_ref,
                  o_ref):
    ssum = parts_ref[0] + parts_ref[1]
    cnt = cnt_ref[0] + cnt_ref[1]
    mean = ssum / jnp.maximum(cnt, 1.0)
    out = lax.dot_general(mean, wl_ref[...], (((1,), (1,)), ((), ())),
                          preferred_element_type=jnp.float32)
    out = out + bl_ref[...]
    out = out + lax.dot_general(x_ref[...], wr_ref[...],
                                (((1,), (1,)), ((), ())),
                                preferred_element_type=jnp.float32)
    nrm = jnp.sqrt(jnp.sum(out * out, axis=1, keepdims=True))
    out = out / jnp.maximum(nrm, 1e-12)
    if mode == 1:
        out = jnp.maximum(out, 0.0)
    else:
        m = jnp.max(out, axis=1, keepdims=True)
        sh = out - m
        out = sh - jnp.log(jnp.sum(jnp.exp(sh), axis=1, keepdims=True))
    o_ref[...] = out


def _combine(parts, cnt, x, w_l, b_l, w_r, mode, interpret=False):
    return pl.pallas_call(
        functools.partial(_combine_body, mode),
        grid=(GRID,),
        in_specs=[
            pl.BlockSpec((2, B, F), lambda i: (0, i, 0)),
            pl.BlockSpec((2, B, 1), lambda i: (0, i, 0)),
            pl.BlockSpec((B, F), lambda i: (i, 0)),
            pl.BlockSpec((F, F), lambda i: (0, 0)),
            pl.BlockSpec((1, F), lambda i: (0, 0)),
            pl.BlockSpec((F, F), lambda i: (0, 0)),
        ],
        out_specs=pl.BlockSpec((B, F), lambda i: (i, 0)),
        out_shape=jax.ShapeDtypeStruct((N_NODES, F), jnp.float32),
        interpret=interpret,
    )(parts, cnt, x, w_l, b_l, w_r)


@functools.lru_cache(maxsize=None)
def _get_agg(with_cnt):
    return _make_agg(with_cnt)


def kernel(x, edge_index, W1_l, b1_l, W1_r, W2_l, b2_l, W2_r):
    _agg_cnt = _get_agg(True)
    _agg = _get_agg(False)
    src = edge_index[0].astype(jnp.int32)
    dst = edge_index[1].astype(jnp.int32)

    pad = E_PAD - N_EDGES
    ar = jnp.arange(pad, dtype=jnp.int32)
    src_p = jnp.concatenate([src, ar % N_NODES])
    dst_p = jnp.concatenate([dst, N_NODES + ar % PAD_DST_ROWS])

    parts1, cnt_flat = _agg_cnt(x, src_p, dst_p)
    cnt = cnt_flat.reshape(2, CNT_PAD, 1)
    p1 = parts1.reshape(2, N_NODES, F)
    h = _combine(p1, cnt, x, W1_l, b1_l.reshape(1, F), W1_r, mode=1)

    parts2 = _agg(h, src_p, dst_p)
    p2 = parts2.reshape(2, N_NODES, F)
    return _combine(p2, cnt, h, W2_l, b2_l.reshape(1, F), W2_r, mode=2)


# speedup vs baseline: 1.2302x; 1.2302x over previous
import functools

import jax
import jax.numpy as jnp
from jax import lax
from jax.experimental import pallas as pl
from jax.experimental.pallas import tpu as pltpu
from jax.experimental.pallas import tpu_sc as plsc

N_NODES = 10000
F = 128
N_EDGES = 320000

NC = 2
NS = 16
NW = NC * NS
K = 128
CPW = 80
E_PAD = NW * CPW * K
PAD_DST_ROWS = 240

ACC_ROWS = 10240
ZPT = ACC_ROWS // NS
CNT_PAD = 10240
CPT = CNT_PAD // NS


def _agg_body(with_cnt, *refs):
    if with_cnt:
        (x_hbm, src_hbm, dst_hbm, out_hbm, cnt_hbm,
         acc_sh, cnt_sh, zflat, src1d, dstv0, dstv1, rows,
         onesv, gsem0, gsem1, dsem0, dsem1) = refs
    else:
        (x_hbm, src_hbm, dst_hbm, out_hbm,
         acc_sh, src1d, dstv0, dstv1, rows, gsem0, gsem1, dsem0,
         dsem1) = refs

    c = lax.axis_index("c")
    s = lax.axis_index("s")
    wid = c * NS + s
    ebase = wid * CPW * K

    pltpu.sync_copy(src_hbm.at[pl.ds(ebase, CPW * K)], src1d)

    zero16 = jnp.zeros((16,), jnp.float32)

    def _zrow(i, carry):
        for jj in range(8):
            rows[0, i, pl.ds(jj * 16, 16)] = zero16
        return carry

    lax.fori_loop(0, K, _zrow, 0)

    for b in range(ZPT // K):
        pltpu.sync_copy(rows.at[0],
                        acc_sh.at[pl.ds(s * ZPT + b * K, K)])

    if with_cnt:
        def _zflat(i, carry):
            zflat[pl.ds(i * 16, 16)] = zero16
            return carry

        lax.fori_loop(0, CPT // 16, _zflat, 0)
        pltpu.sync_copy(zflat, cnt_sh.at[pl.ds(s * CPT, CPT)])
        one16 = jnp.ones((16,), jnp.float32)
        for jj in range(K // 16):
            onesv[pl.ds(jj * 16, 16)] = one16

    def _start(j, b, dref, gsem, dsem):
        pltpu.async_copy(dst_hbm.at[pl.ds(ebase + j * K, K)], dref, dsem)
        pltpu.async_copy(x_hbm.at[src1d.at[pl.ds(j * K, K)]], rows.at[b],
                         gsem)

    _start(0, 0, dstv0, gsem0, dsem0)
    _start(1, 1, dstv1, gsem1, dsem1)

    plsc.subcore_barrier()

    def _do(jd, b, dref, gsem, dsem):
        pltpu.make_async_copy(dst_hbm.at[pl.ds(ebase + jd * K, K)], dref,
                              dsem).wait()
        pltpu.make_async_copy(x_hbm.at[src1d.at[pl.ds(jd * K, K)]],
                              rows.at[b], gsem).wait()
        pltpu.sync_copy(rows.at[b], acc_sh.at[dref], add=True)
        if with_cnt:
            pltpu.sync_copy(onesv, cnt_sh.at[dref], add=True)

    def _pair(i, carry):
        j0 = i * 2
        _do(j0, 0, dstv0, gsem0, dsem0)

        @pl.when(j0 + 2 < CPW)
        def _n0():
            _start(j0 + 2, 0, dstv0, gsem0, dsem0)

        _do(j0 + 1, 1, dstv1, gsem1, dsem1)

        @pl.when(j0 + 3 < CPW)
        def _n1():
            _start(j0 + 3, 1, dstv1, gsem1, dsem1)

        return carry

    lax.fori_loop(0, CPW // 2, _pair, 0)

    plsc.subcore_barrier()

    obase = c * N_NODES + s * ZPT

    @pl.when(s < NS - 1)
    def _copy_full():
        for b in range(ZPT // 128):
            pltpu.sync_copy(acc_sh.at[pl.ds(s * ZPT + b * 128, 128)],
                            out_hbm.at[pl.ds(obase + b * 128, 128)])

    @pl.when(s == NS - 1)
    def _copy_last():
        rem = N_NODES - (NS - 1) * ZPT
        for b in range(rem // 128):
            pltpu.sync_copy(acc_sh.at[pl.ds(s * ZPT + b * 128, 128)],
                            out_hbm.at[pl.ds(obase + b * 128, 128)])
        r = rem - (rem // 128) * 128
        if r:
            pltpu.sync_copy(
                acc_sh.at[pl.ds(s * ZPT + (rem // 128) * 128, r)],
                out_hbm.at[pl.ds(obase + (rem // 128) * 128, r)])
    if with_cnt:
        pltpu.sync_copy(cnt_sh.at[pl.ds(s * CPT, CPT)],
                        cnt_hbm.at[pl.ds(c * CNT_PAD + s * CPT, CPT)])


def _make_agg(with_cnt):
    mesh = plsc.VectorSubcoreMesh(core_axis_name="c", subcore_axis_name="s",
                                  num_cores=NC, num_subcores=NS)
    out_type = [jax.ShapeDtypeStruct((2 * N_NODES, F), jnp.float32)]
    scratch = [
        pltpu.VMEM_SHARED((ACC_ROWS, F), jnp.float32),
    ]
    if with_cnt:
        out_type.append(jax.ShapeDtypeStruct((2 * CNT_PAD,), jnp.float32))
        scratch.append(pltpu.VMEM_SHARED((CNT_PAD,), jnp.float32))
    if with_cnt:
        scratch.append(pltpu.VMEM((CPT,), jnp.float32))
    scratch += [
        pltpu.VMEM((CPW * K,), jnp.int32),
        pltpu.VMEM((K,), jnp.int32),
        pltpu.VMEM((K,), jnp.int32),
        pltpu.VMEM((2, K, F), jnp.float32),
    ]
    if with_cnt:
        scratch.append(pltpu.VMEM((K,), jnp.float32))
    scratch += [pltpu.SemaphoreType.DMA] * 4
    return pl.kernel(functools.partial(_agg_body, with_cnt),
                     out_type=tuple(out_type) if with_cnt else out_type[0],
                     mesh=mesh, scratch_types=scratch)


B = 1000
GRID = N_NODES // B


def _combine_body(mode, parts_ref, cnt_ref, x_ref, wl_ref, bl_ref, wr_ref,
                  o_ref):
    ssum = parts_ref[0] + parts_ref[1]
    cnt = cnt_ref[0] + cnt_ref[1]
    mean = ssum / jnp.maximum(cnt, 1.0)
    out = lax.dot_general(mean, wl_ref[...], (((1,), (1,)), ((), ())),
                          preferred_element_type=jnp.float32)
    out = out + bl_ref[...]
    out = out + lax.dot_general(x_ref[...], wr_ref[...],
                                (((1,), (1,)), ((), ())),
                                preferred_element_type=jnp.float32)
    nrm = jnp.sqrt(jnp.sum(out * out, axis=1, keepdims=True))
    out = out / jnp.maximum(nrm, 1e-12)
    if mode == 1:
        out = jnp.maximum(out, 0.0)
    else:
        m = jnp.max(out, axis=1, keepdims=True)
        sh = out - m
        out = sh - jnp.log(jnp.sum(jnp.exp(sh), axis=1, keepdims=True))
    o_ref[...] = out


def _combine(parts, cnt, x, w_l, b_l, w_r, mode, interpret=False):
    return pl.pallas_call(
        functools.partial(_combine_body, mode),
        grid=(GRID,),
        in_specs=[
            pl.BlockSpec((2, B, F), lambda i: (0, i, 0)),
            pl.BlockSpec((2, B, 1), lambda i: (0, i, 0)),
            pl.BlockSpec((B, F), lambda i: (i, 0)),
            pl.BlockSpec((F, F), lambda i: (0, 0)),
            pl.BlockSpec((1, F), lambda i: (0, 0)),
            pl.BlockSpec((F, F), lambda i: (0, 0)),
        ],
        out_specs=pl.BlockSpec((B, F), lambda i: (i, 0)),
        out_shape=jax.ShapeDtypeStruct((N_NODES, F), jnp.float32),
        interpret=interpret,
    )(parts, cnt, x, w_l, b_l, w_r)


@functools.lru_cache(maxsize=None)
def _get_agg(with_cnt):
    return _make_agg(with_cnt)


def kernel(x, edge_index, W1_l, b1_l, W1_r, W2_l, b2_l, W2_r):
    _agg_cnt = _get_agg(True)
    _agg = _get_agg(False)
    src = edge_index[0].astype(jnp.int32)
    dst = edge_index[1].astype(jnp.int32)

    pad = E_PAD - N_EDGES
    ar = jnp.arange(pad, dtype=jnp.int32)
    src_p = jnp.concatenate([src, ar % N_NODES])
    dst_p = jnp.concatenate([dst, N_NODES + ar % PAD_DST_ROWS])

    parts1, cnt_flat = _agg_cnt(x, src_p, dst_p)
    cnt = cnt_flat.reshape(2, CNT_PAD)[:, :N_NODES].reshape(2, N_NODES, 1)
    p1 = parts1.reshape(2, N_NODES, F)
    h = _combine(p1, cnt, x, W1_l, b1_l.reshape(1, F), W1_r, mode=1)

    parts2 = _agg(h, src_p, dst_p)
    p2 = parts2.reshape(2, N_NODES, F)
    return _combine(p2, cnt, h, W2_l, b2_l.reshape(1, F), W2_r, mode=2)


# speedup vs baseline: 1.2489x; 1.0152x over previous
import functools

import jax
import jax.numpy as jnp
from jax import lax
from jax.experimental import pallas as pl
from jax.experimental.pallas import tpu as pltpu
from jax.experimental.pallas import tpu_sc as plsc

N_NODES = 10000
F = 128
N_EDGES = 320000

NC = 2
NS = 16
NW = NC * NS
K = 128
CPW = 80
E_PAD = NW * CPW * K
PAD_DST_ROWS = 240

ACC_ROWS = 10240
ZPT = ACC_ROWS // NS
CNT_PAD = 10240
CPT = CNT_PAD // NS


def _agg_body(with_cnt, *refs):
    if with_cnt:
        (x_hbm, src_hbm, dst_hbm, out_hbm, cnt_hbm,
         acc_sh, cnt_sh, zflat, src1d, dstv0, dstv1, rows,
         onesv, gsem0, gsem1, dsem0, dsem1) = refs
    else:
        (x_hbm, src_hbm, dst_hbm, out_hbm,
         acc_sh, src1d, dstv0, dstv1, rows, gsem0, gsem1, dsem0,
         dsem1) = refs

    c = lax.axis_index("c")
    s = lax.axis_index("s")
    wid = c * NS + s
    ebase = wid * CPW * K

    pltpu.sync_copy(src_hbm.at[pl.ds(ebase, CPW * K)], src1d)

    zero16 = jnp.zeros((16,), jnp.float32)

    def _zrow(i, carry):
        for jj in range(8):
            rows[0, i, pl.ds(jj * 16, 16)] = zero16
        return carry

    lax.fori_loop(0, K, _zrow, 0)

    for b in range(ZPT // K):
        pltpu.sync_copy(rows.at[0],
                        acc_sh.at[pl.ds(s * ZPT + b * K, K)])

    if with_cnt:
        def _zflat(i, carry):
            zflat[pl.ds(i * 16, 16)] = zero16
            return carry

        lax.fori_loop(0, CPT // 16, _zflat, 0)
        pltpu.sync_copy(zflat, cnt_sh.at[pl.ds(s * CPT, CPT)])
        one16 = jnp.ones((16,), jnp.float32)
        for jj in range(K // 16):
            onesv[pl.ds(jj * 16, 16)] = one16

    def _start(j, b, dref, gsem, dsem):
        pltpu.async_copy(dst_hbm.at[pl.ds(ebase + j * K, K)], dref, dsem)
        pltpu.async_copy(x_hbm.at[src1d.at[pl.ds(j * K, K)]], rows.at[b],
                         gsem)

    _start(0, 0, dstv0, gsem0, dsem0)
    _start(1, 1, dstv1, gsem1, dsem1)

    plsc.subcore_barrier()

    def _do(jd, b, dref, gsem, dsem):
        pltpu.make_async_copy(dst_hbm.at[pl.ds(ebase + jd * K, K)], dref,
                              dsem).wait()
        pltpu.make_async_copy(x_hbm.at[src1d.at[pl.ds(jd * K, K)]],
                              rows.at[b], gsem).wait()
        pltpu.sync_copy(rows.at[b], acc_sh.at[dref], add=True)
        if with_cnt:
            pltpu.sync_copy(onesv, cnt_sh.at[dref], add=True)

    def _pair(i, carry):
        j0 = i * 2
        _do(j0, 0, dstv0, gsem0, dsem0)

        @pl.when(j0 + 2 < CPW)
        def _n0():
            _start(j0 + 2, 0, dstv0, gsem0, dsem0)

        _do(j0 + 1, 1, dstv1, gsem1, dsem1)

        @pl.when(j0 + 3 < CPW)
        def _n1():
            _start(j0 + 3, 1, dstv1, gsem1, dsem1)

        return carry

    lax.fori_loop(0, CPW // 2, _pair, 0)

    plsc.subcore_barrier()

    obase = c * N_NODES + s * ZPT

    @pl.when(s < NS - 1)
    def _copy_full():
        for b in range(ZPT // 128):
            pltpu.sync_copy(acc_sh.at[pl.ds(s * ZPT + b * 128, 128)],
                            out_hbm.at[pl.ds(obase + b * 128, 128)])

    @pl.when(s == NS - 1)
    def _copy_last():
        rem = N_NODES - (NS - 1) * ZPT
        for b in range(rem // 128):
            pltpu.sync_copy(acc_sh.at[pl.ds(s * ZPT + b * 128, 128)],
                            out_hbm.at[pl.ds(obase + b * 128, 128)])
        r = rem - (rem // 128) * 128
        if r:
            pltpu.sync_copy(
                acc_sh.at[pl.ds(s * ZPT + (rem // 128) * 128, r)],
                out_hbm.at[pl.ds(obase + (rem // 128) * 128, r)])
    if with_cnt:
        pltpu.sync_copy(cnt_sh.at[pl.ds(s * CPT, CPT)],
                        cnt_hbm.at[pl.ds(c * CNT_PAD + s * CPT, CPT)])


def _make_agg(with_cnt):
    mesh = plsc.VectorSubcoreMesh(core_axis_name="c", subcore_axis_name="s",
                                  num_cores=NC, num_subcores=NS)
    out_type = [jax.ShapeDtypeStruct((2 * N_NODES, F), jnp.float32)]
    scratch = [
        pltpu.VMEM_SHARED((ACC_ROWS, F), jnp.float32),
    ]
    if with_cnt:
        out_type.append(jax.ShapeDtypeStruct((2 * CNT_PAD,), jnp.float32))
        scratch.append(pltpu.VMEM_SHARED((CNT_PAD,), jnp.float32))
    if with_cnt:
        scratch.append(pltpu.VMEM((CPT,), jnp.float32))
    scratch += [
        pltpu.VMEM((CPW * K,), jnp.int32),
        pltpu.VMEM((K,), jnp.int32),
        pltpu.VMEM((K,), jnp.int32),
        pltpu.VMEM((2, K, F), jnp.float32),
    ]
    if with_cnt:
        scratch.append(pltpu.VMEM((K,), jnp.float32))
    scratch += [pltpu.SemaphoreType.DMA] * 4
    return pl.kernel(functools.partial(_agg_body, with_cnt),
                     out_type=tuple(out_type) if with_cnt else out_type[0],
                     mesh=mesh, scratch_types=scratch)


B = 2000
GRID = N_NODES // B


def _combine_body(mode, parts_ref, cnt_ref, x_ref, wl_ref, bl_ref, wr_ref,
                  o_ref):
    ssum = parts_ref[0] + parts_ref[1]
    cnt = cnt_ref[0] + cnt_ref[1]
    mean = ssum / jnp.maximum(cnt, 1.0)
    out = lax.dot_general(mean, wl_ref[...], (((1,), (1,)), ((), ())),
                          preferred_element_type=jnp.float32)
    out = out + bl_ref[...]
    out = out + lax.dot_general(x_ref[...], wr_ref[...],
                                (((1,), (1,)), ((), ())),
                                preferred_element_type=jnp.float32)
    nrm = jnp.sqrt(jnp.sum(out * out, axis=1, keepdims=True))
    out = out / jnp.maximum(nrm, 1e-12)
    if mode == 1:
        out = jnp.maximum(out, 0.0)
    else:
        m = jnp.max(out, axis=1, keepdims=True)
        sh = out - m
        out = sh - jnp.log(jnp.sum(jnp.exp(sh), axis=1, keepdims=True))
    o_ref[...] = out


def _combine(parts, cnt, x, w_l, b_l, w_r, mode, interpret=False):
    return pl.pallas_call(
        functools.partial(_combine_body, mode),
        grid=(GRID,),
        in_specs=[
            pl.BlockSpec((2, B, F), lambda i: (0, i, 0)),
            pl.BlockSpec((2, B, 1), lambda i: (0, i, 0)),
            pl.BlockSpec((B, F), lambda i: (i, 0)),
            pl.BlockSpec((F, F), lambda i: (0, 0)),
            pl.BlockSpec((1, F), lambda i: (0, 0)),
            pl.BlockSpec((F, F), lambda i: (0, 0)),
        ],
        out_specs=pl.BlockSpec((B, F), lambda i: (i, 0)),
        out_shape=jax.ShapeDtypeStruct((N_NODES, F), jnp.float32),
        interpret=interpret,
    )(parts, cnt, x, w_l, b_l, w_r)


@functools.lru_cache(maxsize=None)
def _get_agg(with_cnt):
    return _make_agg(with_cnt)


def kernel(x, edge_index, W1_l, b1_l, W1_r, W2_l, b2_l, W2_r):
    _agg_cnt = _get_agg(True)
    _agg = _get_agg(False)
    src = edge_index[0].astype(jnp.int32)
    dst = edge_index[1].astype(jnp.int32)

    pad = E_PAD - N_EDGES
    ar = jnp.arange(pad, dtype=jnp.int32)
    src_p = jnp.concatenate([src, ar % N_NODES])
    dst_p = jnp.concatenate([dst, N_NODES + ar % PAD_DST_ROWS])

    parts1, cnt_flat = _agg_cnt(x, src_p, dst_p)
    cnt = cnt_flat.reshape(2, CNT_PAD)[:, :N_NODES].reshape(2, N_NODES, 1)
    p1 = parts1.reshape(2, N_NODES, F)
    h = _combine(p1, cnt, x, W1_l, b1_l.reshape(1, F), W1_r, mode=1)

    parts2 = _agg(h, src_p, dst_p)
    p2 = parts2.reshape(2, N_NODES, F)
    return _combine(p2, cnt, h, W2_l, b2_l.reshape(1, F), W2_r, mode=2)


# speedup vs baseline: 1.2587x; 1.0078x over previous
import functools

import jax
import jax.numpy as jnp
from jax import lax
from jax.experimental import pallas as pl
from jax.experimental.pallas import tpu as pltpu
from jax.experimental.pallas import tpu_sc as plsc

N_NODES = 10000
F = 128
N_EDGES = 320000

NC = 2
NS = 16
NW = NC * NS
K = 128
CPW = 80
E_PAD = NW * CPW * K
PAD_DST_ROWS = 240

ACC_ROWS = 10240
ZPT = ACC_ROWS // NS
CNT_PAD = 10240
CPT = CNT_PAD // NS


def _agg_body(with_cnt, *refs):
    if with_cnt:
        (x_hbm, src_hbm, dst_hbm, out_hbm, cnt_hbm,
         acc_sh, cnt_sh, zflat, src1d, dstv0, dstv1, rows,
         onesv, gsem0, gsem1, dsem0, dsem1) = refs
    else:
        (x_hbm, src_hbm, dst_hbm, out_hbm,
         acc_sh, src1d, dstv0, dstv1, rows, gsem0, gsem1, dsem0,
         dsem1) = refs

    c = lax.axis_index("c")
    s = lax.axis_index("s")
    wid = c * NS + s
    ebase = wid * CPW * K

    pltpu.sync_copy(src_hbm.at[pl.ds(ebase, CPW * K)], src1d)

    zero16 = jnp.zeros((16,), jnp.float32)

    def _zrow(i, carry):
        for jj in range(8):
            rows[0, i, pl.ds(jj * 16, 16)] = zero16
        return carry

    lax.fori_loop(0, K, _zrow, 0)

    for b in range(ZPT // K):
        pltpu.sync_copy(rows.at[0],
                        acc_sh.at[pl.ds(s * ZPT + b * K, K)])

    if with_cnt:
        def _zflat(i, carry):
            zflat[pl.ds(i * 16, 16)] = zero16
            return carry

        lax.fori_loop(0, CPT // 16, _zflat, 0)
        pltpu.sync_copy(zflat, cnt_sh.at[pl.ds(s * CPT, CPT)])
        one16 = jnp.ones((16,), jnp.float32)
        for jj in range(K // 16):
            onesv[pl.ds(jj * 16, 16)] = one16

    def _start(j, b, dref, gsem, dsem):
        pltpu.async_copy(dst_hbm.at[pl.ds(ebase + j * K, K)], dref, dsem)
        pltpu.async_copy(x_hbm.at[src1d.at[pl.ds(j * K, K)]], rows.at[b],
                         gsem)

    _start(0, 0, dstv0, gsem0, dsem0)
    _start(1, 1, dstv1, gsem1, dsem1)

    plsc.subcore_barrier()

    def _do(jd, b, dref, gsem, dsem):
        pltpu.make_async_copy(dst_hbm.at[pl.ds(ebase + jd * K, K)], dref,
                              dsem).wait()
        pltpu.make_async_copy(x_hbm.at[src1d.at[pl.ds(jd * K, K)]],
                              rows.at[b], gsem).wait()
        pltpu.sync_copy(rows.at[b], acc_sh.at[dref], add=True)
        if with_cnt:
            pltpu.sync_copy(onesv, cnt_sh.at[dref], add=True)

    def _pair(i, carry):
        j0 = i * 2
        _do(j0, 0, dstv0, gsem0, dsem0)

        @pl.when(j0 + 2 < CPW)
        def _n0():
            _start(j0 + 2, 0, dstv0, gsem0, dsem0)

        _do(j0 + 1, 1, dstv1, gsem1, dsem1)

        @pl.when(j0 + 3 < CPW)
        def _n1():
            _start(j0 + 3, 1, dstv1, gsem1, dsem1)

        return carry

    lax.fori_loop(0, CPW // 2, _pair, 0)

    plsc.subcore_barrier()

    obase = c * N_NODES + s * ZPT

    @pl.when(s < NS - 1)
    def _copy_full():
        for b in range(ZPT // 128):
            pltpu.sync_copy(acc_sh.at[pl.ds(s * ZPT + b * 128, 128)],
                            out_hbm.at[pl.ds(obase + b * 128, 128)])

    @pl.when(s == NS - 1)
    def _copy_last():
        rem = N_NODES - (NS - 1) * ZPT
        for b in range(rem // 128):
            pltpu.sync_copy(acc_sh.at[pl.ds(s * ZPT + b * 128, 128)],
                            out_hbm.at[pl.ds(obase + b * 128, 128)])
        r = rem - (rem // 128) * 128
        if r:
            pltpu.sync_copy(
                acc_sh.at[pl.ds(s * ZPT + (rem // 128) * 128, r)],
                out_hbm.at[pl.ds(obase + (rem // 128) * 128, r)])
    if with_cnt:
        pltpu.sync_copy(cnt_sh.at[pl.ds(s * CPT, CPT)],
                        cnt_hbm.at[pl.ds(c * CNT_PAD + s * CPT, CPT)])


def _make_agg(with_cnt):
    mesh = plsc.VectorSubcoreMesh(core_axis_name="c", subcore_axis_name="s",
                                  num_cores=NC, num_subcores=NS)
    out_type = [jax.ShapeDtypeStruct((2 * N_NODES, F), jnp.float32)]
    scratch = [
        pltpu.VMEM_SHARED((ACC_ROWS, F), jnp.float32),
    ]
    if with_cnt:
        out_type.append(jax.ShapeDtypeStruct((2 * CNT_PAD,), jnp.float32))
        scratch.append(pltpu.VMEM_SHARED((CNT_PAD,), jnp.float32))
    if with_cnt:
        scratch.append(pltpu.VMEM((CPT,), jnp.float32))
    scratch += [
        pltpu.VMEM((CPW * K,), jnp.int32),
        pltpu.VMEM((K,), jnp.int32),
        pltpu.VMEM((K,), jnp.int32),
        pltpu.VMEM((2, K, F), jnp.float32),
    ]
    if with_cnt:
        scratch.append(pltpu.VMEM((K,), jnp.float32))
    scratch += [pltpu.SemaphoreType.DMA] * 4
    return pl.kernel(functools.partial(_agg_body, with_cnt),
                     out_type=tuple(out_type) if with_cnt else out_type[0],
                     mesh=mesh, scratch_types=scratch)


B = 5000
GRID = N_NODES // B


def _combine_body(mode, parts_ref, cnt_ref, x_ref, wl_ref, bl_ref, wr_ref,
                  o_ref):
    ssum = parts_ref[0] + parts_ref[1]
    cnt = cnt_ref[0] + cnt_ref[1]
    mean = ssum / jnp.maximum(cnt, 1.0)
    out = lax.dot_general(mean, wl_ref[...], (((1,), (1,)), ((), ())),
                          preferred_element_type=jnp.float32)
    out = out + bl_ref[...]
    out = out + lax.dot_general(x_ref[...], wr_ref[...],
                                (((1,), (1,)), ((), ())),
                                preferred_element_type=jnp.float32)
    nrm = jnp.sqrt(jnp.sum(out * out, axis=1, keepdims=True))
    out = out / jnp.maximum(nrm, 1e-12)
    if mode == 1:
        out = jnp.maximum(out, 0.0)
    else:
        m = jnp.max(out, axis=1, keepdims=True)
        sh = out - m
        out = sh - jnp.log(jnp.sum(jnp.exp(sh), axis=1, keepdims=True))
    o_ref[...] = out


def _combine(parts, cnt, x, w_l, b_l, w_r, mode, interpret=False):
    return pl.pallas_call(
        functools.partial(_combine_body, mode),
        grid=(GRID,),
        in_specs=[
            pl.BlockSpec((2, B, F), lambda i: (0, i, 0)),
            pl.BlockSpec((2, B, 1), lambda i: (0, i, 0)),
            pl.BlockSpec((B, F), lambda i: (i, 0)),
            pl.BlockSpec((F, F), lambda i: (0, 0)),
            pl.BlockSpec((1, F), lambda i: (0, 0)),
            pl.BlockSpec((F, F), lambda i: (0, 0)),
        ],
        out_specs=pl.BlockSpec((B, F), lambda i: (i, 0)),
        out_shape=jax.ShapeDtypeStruct((N_NODES, F), jnp.float32),
        interpret=interpret,
    )(parts, cnt, x, w_l, b_l, w_r)


@functools.lru_cache(maxsize=None)
def _get_agg(with_cnt):
    return _make_agg(with_cnt)


def kernel(x, edge_index, W1_l, b1_l, W1_r, W2_l, b2_l, W2_r):
    _agg_cnt = _get_agg(True)
    _agg = _get_agg(False)
    src = edge_index[0].astype(jnp.int32)
    dst = edge_index[1].astype(jnp.int32)

    pad = E_PAD - N_EDGES
    ar = jnp.arange(pad, dtype=jnp.int32)
    src_p = jnp.concatenate([src, ar % N_NODES])
    dst_p = jnp.concatenate([dst, N_NODES + ar % PAD_DST_ROWS])

    parts1, cnt_flat = _agg_cnt(x, src_p, dst_p)
    cnt = cnt_flat.reshape(2, CNT_PAD)[:, :N_NODES].reshape(2, N_NODES, 1)
    p1 = parts1.reshape(2, N_NODES, F)
    h = _combine(p1, cnt, x, W1_l, b1_l.reshape(1, F), W1_r, mode=1)

    parts2 = _agg(h, src_p, dst_p)
    p2 = parts2.reshape(2, N_NODES, F)
    return _combine(p2, cnt, h, W2_l, b2_l.reshape(1, F), W2_r, mode=2)
